# baseline (speedup 1.0000x reference)
import jax
import jax.numpy as jnp
from jax import lax
from jax.experimental import pallas as pl

NUM_FRAMES = 1024
NUM_TYPES = 4
NUM_ACTIONS = 128
TOTAL_CLS = 256
FB = 128


def _onehot_body(a_ref, o_ref):
    a = a_ref[...]
    toff = lax.broadcasted_iota(jnp.int32, (FB, NUM_TYPES, NUM_ACTIONS), 1)
    a2 = (a + (toff << 6)).astype(jnp.float32)
    t_io = lax.broadcasted_iota(jnp.int32, (NUM_TYPES, TOTAL_CLS), 0)
    c_io = lax.broadcasted_iota(jnp.int32, (NUM_TYPES, TOTAL_CLS), 1)
    p = (t_io == (c_io >> 6)).astype(jnp.float32)
    k = lax.dot_general(
        a2, p, (((1,), (0,)), ((), ())), preferred_element_type=jnp.float32
    )
    col = lax.broadcasted_iota(
        jnp.int32, (FB, NUM_ACTIONS, TOTAL_CLS), 2
    ).astype(jnp.float32)
    o_ref[...] = (k == col).astype(jnp.float32)


def kernel(actions):
    grid = (NUM_FRAMES // FB,)
    return pl.pallas_call(
        _onehot_body,
        grid=grid,
        in_specs=[
            pl.BlockSpec((FB, NUM_TYPES, NUM_ACTIONS), lambda i: (i, 0, 0))
        ],
        out_specs=pl.BlockSpec(
            (FB, NUM_ACTIONS, TOTAL_CLS), lambda i: (i, 0, 0)
        ),
        out_shape=jax.ShapeDtypeStruct(
            (NUM_FRAMES, NUM_ACTIONS, TOTAL_CLS), jnp.float32
        ),
    )(actions)
